# Initial kernel scaffold; baseline (speedup 1.0000x reference)
import jax
import jax.numpy as jnp
from jax.experimental import pallas as pl


def kernel(lang, batch_input, table):
    raise NotImplementedError("write your pallas kernel here")


# speedup vs baseline: 1.7940x; 1.7940x over previous
import functools

import jax
import jax.numpy as jnp
from jax import lax
from jax.experimental import pallas as pl
from jax.experimental.pallas import tpu as pltpu
from jax.experimental.pallas import tpu_sc as plsc

_NC = 2
_NS = 16
_NW = _NC * _NS

_BLK = 512


def _make_gather(total_rows: int, embed_dim: int):
    per_w = total_rows // _NW
    n_blk = per_w // _BLK
    mesh = plsc.VectorSubcoreMesh(core_axis_name="c", subcore_axis_name="s")

    @functools.partial(
        pl.kernel,
        mesh=mesh,
        out_type=jax.ShapeDtypeStruct((total_rows, embed_dim), jnp.float32),
        compiler_params=pltpu.CompilerParams(use_tc_tiling_on_sc=False),
        scratch_types=[
            pltpu.VMEM((_BLK,), jnp.int32),
            pltpu.VMEM((_BLK, embed_dim), jnp.float32),
            pltpu.SemaphoreType.DMA,
        ],
    )
    def gather_kernel(idx_hbm, table_hbm, out_hbm, idx_v, rows_v, sem):
        wid = lax.axis_index("s") * _NC + lax.axis_index("c")
        base = wid * per_w

        def step(i, carry):
            b = base + i * _BLK
            pltpu.sync_copy(idx_hbm.at[pl.ds(b, _BLK)], idx_v)
            pltpu.async_copy(table_hbm.at[idx_v], rows_v, sem).wait()
            pltpu.sync_copy(rows_v, out_hbm.at[pl.ds(b, _BLK)])
            return carry

        lax.fori_loop(0, n_blk, step, 0)

    return gather_kernel


def kernel(lang, batch_input, table):
    del lang
    bsz, seq = batch_input.shape
    _, embed_dim = table.shape
    idx = batch_input.reshape(-1)
    out = _make_gather(bsz * seq, embed_dim)(idx, table)
    return out.reshape(bsz, seq, embed_dim)


# speedup vs baseline: 1.8861x; 1.0513x over previous
import functools

import jax
import jax.numpy as jnp
from jax import lax
from jax.experimental import pallas as pl
from jax.experimental.pallas import tpu as pltpu
from jax.experimental.pallas import tpu_sc as plsc

_NC = 2
_NS = 16
_NW = _NC * _NS

_BLK = 640


def _make_gather(total_rows: int, embed_dim: int):
    per_w = total_rows // _NW
    n_blk = per_w // _BLK
    assert n_blk % 2 == 0
    mesh = plsc.VectorSubcoreMesh(core_axis_name="c", subcore_axis_name="s")

    @functools.partial(
        pl.kernel,
        mesh=mesh,
        out_type=jax.ShapeDtypeStruct((total_rows, embed_dim), jnp.float32),
        compiler_params=pltpu.CompilerParams(use_tc_tiling_on_sc=False),
        scratch_types=[
            pltpu.VMEM((per_w,), jnp.int32),
            pltpu.VMEM((2, _BLK, embed_dim), jnp.float32),
            pltpu.SemaphoreType.DMA,
            pltpu.SemaphoreType.DMA,
            pltpu.SemaphoreType.DMA,
            pltpu.SemaphoreType.DMA,
        ],
    )
    def gather_kernel(idx_hbm, table_hbm, out_hbm, idx_v, rows_v, g0, g1, s0, s1):
        wid = lax.axis_index("s") * _NC + lax.axis_index("c")
        base = wid * per_w
        gsem = (g0, g1)
        ssem = (s0, s1)

        pltpu.sync_copy(idx_hbm.at[pl.ds(base, per_w)], idx_v)

        def gather_copy(i, b):
            return pltpu.make_async_copy(
                table_hbm.at[idx_v.at[pl.ds(i * _BLK, _BLK)]],
                rows_v.at[b],
                gsem[b],
            )

        def store_copy(i, b):
            return pltpu.make_async_copy(
                rows_v.at[b],
                out_hbm.at[pl.ds(base + i * _BLK, _BLK)],
                ssem[b],
            )

        gather_copy(0, 0).start()
        gather_copy(1, 1).start()

        def step(g, carry):
            for b in (0, 1):
                i = g * 2 + b
                gather_copy(i, b).wait()
                store_copy(i, b).start()
                store_copy(i, b).wait()

                @pl.when(i + 2 < n_blk)
                def _():
                    gather_copy(i + 2, b).start()
            return carry

        lax.fori_loop(0, n_blk // 2, step, 0)

    return gather_kernel


def kernel(lang, batch_input, table):
    del lang
    bsz, seq = batch_input.shape
    _, embed_dim = table.shape
    idx = batch_input.reshape(-1)
    out = _make_gather(bsz * seq, embed_dim)(idx, table)
    return out.reshape(bsz, seq, embed_dim)
